# Initial kernel scaffold; baseline (speedup 1.0000x reference)
import jax
import jax.numpy as jnp
from jax.experimental import pallas as pl


def kernel(x, edge_index, Wl1, Wr1, b1, g1, be1, Wl2, Wr2, b2, g2, be2):
    raise NotImplementedError("write your pallas kernel here")


# speedup vs baseline: 4.7024x; 4.7024x over previous
import functools

import jax
import jax.numpy as jnp
from jax import lax
from jax.experimental import pallas as pl
from jax.experimental.pallas import tpu as pltpu
from jax.experimental.pallas import tpu_sc as plsc

NC = 2
NS = 16
NW = NC * NS
CHUNK = 80

_HIGH = lax.Precision.HIGHEST


def _segment_sum_sc(ylaug, src, dst, zeros):
    _, dp = ylaug.shape
    n = zeros.shape[0]
    e = src.shape[0]
    epw = e // NW
    steps = epw // CHUNK
    rpt = n // NS
    mesh = plsc.VectorSubcoreMesh(
        core_axis_name="c", subcore_axis_name="s", num_cores=NC, num_subcores=NS)

    @functools.partial(
        pl.kernel,
        out_type=jax.ShapeDtypeStruct((NC, n, dp), jnp.float32),
        mesh=mesh,
        scratch_types=[
            pltpu.VMEM((CHUNK,), jnp.int32),
            pltpu.VMEM((CHUNK,), jnp.int32),
            pltpu.VMEM((CHUNK, dp), jnp.float32),
            pltpu.VMEM_SHARED((n, dp), jnp.float32),
        ],
        compiler_params=pltpu.CompilerParams(use_tc_tiling_on_sc=False),
    )
    def segsum(yl_hbm, src_hbm, dst_hbm, z_hbm, out_hbm, src_v, dst_v, rows_v, acc):
        c = lax.axis_index("c")
        s = lax.axis_index("s")
        wid = s * NC + c
        pltpu.sync_copy(z_hbm.at[pl.ds(s * rpt, rpt)], acc.at[pl.ds(s * rpt, rpt)])
        plsc.subcore_barrier()

        def body(i, carry):
            off = wid * epw + i * CHUNK
            pltpu.sync_copy(src_hbm.at[pl.ds(off, CHUNK)], src_v)
            pltpu.sync_copy(dst_hbm.at[pl.ds(off, CHUNK)], dst_v)
            pltpu.sync_copy(yl_hbm.at[src_v], rows_v)
            pltpu.sync_copy(rows_v, acc.at[dst_v], add=True)
            return carry

        lax.fori_loop(0, steps, body, 0)
        plsc.subcore_barrier()
        pltpu.sync_copy(acc.at[pl.ds(s * rpt, rpt)],
                        out_hbm.at[c, pl.ds(s * rpt, rpt)])

    return segsum(ylaug, src, dst, zeros)


def _pre_body(x_ref, wla_ref, wr_ref, b_ref, e_ref, ylaug_ref, yr_ref):
    x = x_ref[...]
    ylaug_ref[...] = (
        jnp.dot(x, wla_ref[...], preferred_element_type=jnp.float32,
                precision=_HIGH) + e_ref[...])
    yr_ref[...] = (
        jnp.dot(x, wr_ref[...], preferred_element_type=jnp.float32,
                precision=_HIGH) + b_ref[...])


def _bn(h0, g_ref, be_ref):
    mu = jnp.mean(h0, axis=0, keepdims=True)
    var = jnp.mean((h0 - mu) * (h0 - mu), axis=0, keepdims=True)
    return (h0 - mu) * lax.rsqrt(var + 1e-5) * g_ref[...] + be_ref[...]


def _bnact_body(n, d, relu, parts_ref, yr_ref, g_ref, be_ref, out_ref):
    agg = parts_ref[0, :n] + parts_ref[1, :n]
    deg = jnp.maximum(agg[:, d:d + 1], 1.0)
    h0 = agg[:, :d] / deg + yr_ref[...]
    h = _bn(h0, g_ref, be_ref)
    out_ref[...] = jnp.maximum(h, 0.0) if relu else h


def kernel(x, edge_index, Wl1, Wr1, b1, g1, be1, Wl2, Wr2, b2, g2, be2):
    n, d = x.shape
    dp = d + 16
    f32 = jnp.float32
    src = edge_index[0]
    dst = edge_index[1]
    ehot = jnp.zeros((1, dp), f32).at[0, d].set(1.0)
    wla1 = jnp.pad(Wl1, ((0, 0), (0, dp - d)))
    wla2 = jnp.pad(Wl2, ((0, 0), (0, dp - d)))
    n_pad = -(-n // (NS * 8)) * (NS * 8)
    zeros = jnp.zeros((n_pad, dp), f32)

    ylaug1, yr1 = pl.pallas_call(
        _pre_body,
        out_shape=[jax.ShapeDtypeStruct((n, dp), f32),
                   jax.ShapeDtypeStruct((n, d), f32)],
    )(x, wla1, Wr1, b1.reshape(1, d), ehot)

    parts1 = _segment_sum_sc(ylaug1, src, dst, zeros)

    h = pl.pallas_call(
        functools.partial(_bnact_body, n, d, True),
        out_shape=jax.ShapeDtypeStruct((n, d), f32),
    )(parts1, yr1, g1.reshape(1, d), be1.reshape(1, d))

    ylaug2, yr2 = pl.pallas_call(
        _pre_body,
        out_shape=[jax.ShapeDtypeStruct((n, dp), f32),
                   jax.ShapeDtypeStruct((n, d), f32)],
    )(h, wla2, Wr2, b2.reshape(1, d), ehot)

    parts2 = _segment_sum_sc(ylaug2, src, dst, zeros)

    out = pl.pallas_call(
        functools.partial(_bnact_body, n, d, False),
        out_shape=jax.ShapeDtypeStruct((n, d), f32),
    )(parts2, yr2, g2.reshape(1, d), be2.reshape(1, d))
    return out


# speedup vs baseline: 7.3438x; 1.5617x over previous
import functools

import jax
import jax.numpy as jnp
from jax import lax
from jax.experimental import pallas as pl
from jax.experimental.pallas import tpu as pltpu
from jax.experimental.pallas import tpu_sc as plsc

NC = 2
NS = 16
NW = NC * NS
CHUNK = 80

_HIGH = lax.Precision.HIGHEST


def _segment_sum_sc(ylaug, src, dst, zeros):
    _, dp = ylaug.shape
    n = zeros.shape[0]
    e = src.shape[0]
    epw = e // NW
    steps = epw // CHUNK
    rpt = n // NS
    mesh = plsc.VectorSubcoreMesh(
        core_axis_name="c", subcore_axis_name="s", num_cores=NC, num_subcores=NS)

    @functools.partial(
        pl.kernel,
        out_type=jax.ShapeDtypeStruct((NC, n, dp), jnp.float32),
        mesh=mesh,
        scratch_types=[
            pltpu.VMEM((2, CHUNK), jnp.int32),
            pltpu.VMEM((2, CHUNK), jnp.int32),
            pltpu.VMEM((CHUNK, dp), jnp.float32),
            pltpu.VMEM((CHUNK, dp), jnp.float32),
            pltpu.VMEM_SHARED((n, dp), jnp.float32),
            pltpu.SemaphoreType.DMA,
            pltpu.SemaphoreType.DMA,
        ],
        compiler_params=pltpu.CompilerParams(use_tc_tiling_on_sc=False),
    )
    def segsum(yl_hbm, sd_hbm, z_hbm, out_hbm,
               idx_a, idx_b, rows_a, rows_b, acc, sem_a, sem_b):
        c = lax.axis_index("c")
        s = lax.axis_index("s")
        wid = s * NC + c
        pltpu.sync_copy(z_hbm.at[pl.ds(s * rpt, rpt)], acc.at[pl.ds(s * rpt, rpt)])
        plsc.subcore_barrier()

        last = steps - 1

        def load_idx(i, idx):
            pltpu.sync_copy(sd_hbm.at[wid, i], idx)

        def gather(idx, rows, sem):
            pltpu.make_async_copy(yl_hbm.at[idx.at[0]], rows, sem).start()

        def wait(rows, sem):
            pltpu.make_async_copy(yl_hbm, rows, sem).wait()

        def scatter(idx, rows):
            pltpu.sync_copy(rows, acc.at[idx.at[1]], add=True)

        load_idx(0, idx_a)
        gather(idx_a, rows_a, sem_a)
        load_idx(jnp.minimum(1, last), idx_b)

        def pair(p, carry):
            i = 2 * p
            wait(rows_a, sem_a)
            gather(idx_b, rows_b, sem_b)
            scatter(idx_a, rows_a)
            load_idx(jnp.minimum(i + 2, last), idx_a)
            wait(rows_b, sem_b)

            @pl.when(i + 2 < steps)
            def _():
                gather(idx_a, rows_a, sem_a)

            scatter(idx_b, rows_b)
            load_idx(jnp.minimum(i + 3, last), idx_b)
            return carry

        lax.fori_loop(0, steps // 2, pair, 0)
        if steps % 2 == 1:
            wait(rows_a, sem_a)
            scatter(idx_a, rows_a)
        plsc.subcore_barrier()
        pltpu.sync_copy(acc.at[pl.ds(s * rpt, rpt)],
                        out_hbm.at[c, pl.ds(s * rpt, rpt)])

    sd = jnp.stack([src.reshape(NW, steps, CHUNK),
                    dst.reshape(NW, steps, CHUNK)], axis=2)
    return segsum(ylaug, sd, zeros)


def _pre_body(x_ref, wla_ref, wr_ref, b_ref, e_ref, ylaug_ref, yr_ref):
    x = x_ref[...]
    ylaug_ref[...] = (
        jnp.dot(x, wla_ref[...], preferred_element_type=jnp.float32,
                precision=_HIGH) + e_ref[...])
    yr_ref[...] = (
        jnp.dot(x, wr_ref[...], preferred_element_type=jnp.float32,
                precision=_HIGH) + b_ref[...])


def _bn(h0, g_ref, be_ref):
    mu = jnp.mean(h0, axis=0, keepdims=True)
    var = jnp.mean((h0 - mu) * (h0 - mu), axis=0, keepdims=True)
    return (h0 - mu) * lax.rsqrt(var + 1e-5) * g_ref[...] + be_ref[...]


def _bnact_body(n, d, relu, parts_ref, yr_ref, g_ref, be_ref, out_ref):
    agg = parts_ref[0, :n] + parts_ref[1, :n]
    deg = jnp.maximum(agg[:, d:d + 1], 1.0)
    h0 = agg[:, :d] / deg + yr_ref[...]
    h = _bn(h0, g_ref, be_ref)
    out_ref[...] = jnp.maximum(h, 0.0) if relu else h


def kernel(x, edge_index, Wl1, Wr1, b1, g1, be1, Wl2, Wr2, b2, g2, be2):
    n, d = x.shape
    dp = d + 16
    f32 = jnp.float32
    src = edge_index[0]
    dst = edge_index[1]
    ehot = jnp.zeros((1, dp), f32).at[0, d].set(1.0)
    wla1 = jnp.pad(Wl1, ((0, 0), (0, dp - d)))
    wla2 = jnp.pad(Wl2, ((0, 0), (0, dp - d)))
    n_pad = -(-n // (NS * 8)) * (NS * 8)
    zeros = jnp.zeros((n_pad, dp), f32)

    ylaug1, yr1 = pl.pallas_call(
        _pre_body,
        out_shape=[jax.ShapeDtypeStruct((n, dp), f32),
                   jax.ShapeDtypeStruct((n, d), f32)],
    )(x, wla1, Wr1, b1.reshape(1, d), ehot)

    parts1 = _segment_sum_sc(ylaug1, src, dst, zeros)

    h = pl.pallas_call(
        functools.partial(_bnact_body, n, d, True),
        out_shape=jax.ShapeDtypeStruct((n, d), f32),
    )(parts1, yr1, g1.reshape(1, d), be1.reshape(1, d))

    ylaug2, yr2 = pl.pallas_call(
        _pre_body,
        out_shape=[jax.ShapeDtypeStruct((n, dp), f32),
                   jax.ShapeDtypeStruct((n, d), f32)],
    )(h, wla2, Wr2, b2.reshape(1, d), ehot)

    parts2 = _segment_sum_sc(ylaug2, src, dst, zeros)

    out = pl.pallas_call(
        functools.partial(_bnact_body, n, d, False),
        out_shape=jax.ShapeDtypeStruct((n, d), f32),
    )(parts2, yr2, g2.reshape(1, d), be2.reshape(1, d))
    return out
